# baseline (device time: 15417 ns/iter reference)
import jax
import jax.numpy as jnp
from jax import lax
from jax.experimental import pallas as pl
from jax.experimental.pallas import tpu as pltpu

N_DEV = 4
N_TOK = 512
D_IN = 256
D_OUT = 512
N_EXP = 8
EXP_PER = N_EXP // N_DEV
CHUNK = N_TOK // N_DEV


def kernel(x, router_W, route_idx, expert_W):
    def body(x_ref, rw_ref, idx_ref, ew_ref, out_ref,
             xw_ref, send_buf, comm_ref, send_sems, recv_sems):
        my = lax.axis_index("i")

        barrier_sem = pltpu.get_barrier_semaphore()
        for k in range(1, N_DEV):
            pl.semaphore_signal(
                barrier_sem, inc=1,
                device_id=(jnp.mod(my + k, N_DEV),),
                device_id_type=pl.DeviceIdType.MESH,
            )

        xv = x_ref[:, :]
        scores = jnp.dot(xv, rw_ref[:, :], preferred_element_type=jnp.float32)
        m = jnp.max(scores, axis=-1, keepdims=True)
        p = jnp.exp(scores - m)
        p = p / jnp.sum(p, axis=-1, keepdims=True)

        idx = idx_ref[:, :]
        cols = lax.broadcasted_iota(jnp.int32, (N_TOK, N_EXP), 1)
        sel0 = idx[:, 0:1] == cols
        sel1 = idx[:, 1:2] == cols
        g0 = jnp.sum(jnp.where(sel0, p, 0.0), axis=1, keepdims=True)
        g1 = jnp.sum(jnp.where(sel1, p, 0.0), axis=1, keepdims=True)
        wfull = jnp.where(sel0 | sel1, p, 0.0) / (g0 + g1)

        e_base = my * EXP_PER
        w0 = jnp.sum(jnp.where(cols == e_base, wfull, 0.0), axis=1, keepdims=True)
        w1 = jnp.sum(jnp.where(cols == e_base + 1, wfull, 0.0), axis=1, keepdims=True)
        xw_ref[:, :D_IN] = w0 * xv
        xw_ref[:, D_IN:] = w1 * xv
        Wcat = jnp.concatenate([ew_ref[0], ew_ref[1]], axis=0)

        pl.semaphore_wait(barrier_sem, N_DEV - 1)

        sends = []
        for k in range(1, N_DEV):
            dst = jnp.mod(my + k, N_DEV)
            rows = pl.ds(dst * CHUNK, CHUNK)
            send_buf[rows, :] = jnp.dot(
                xw_ref[rows, :], Wcat, preferred_element_type=jnp.float32
            )
            rdma = pltpu.make_async_remote_copy(
                src_ref=send_buf.at[rows, :],
                dst_ref=comm_ref.at[pl.ds(my * CHUNK, CHUNK), :],
                send_sem=send_sems.at[k - 1],
                recv_sem=recv_sems.at[my],
                device_id=(dst,),
                device_id_type=pl.DeviceIdType.MESH,
            )
            rdma.start()
            sends.append(rdma)

        my_rows = pl.ds(my * CHUNK, CHUNK)
        own = jnp.dot(xw_ref[my_rows, :], Wcat, preferred_element_type=jnp.float32)

        acc = own
        for k in range(1, N_DEV):
            src = jnp.mod(my - k, N_DEV)
            recv = pltpu.make_async_remote_copy(
                src_ref=send_buf.at[pl.ds(0, CHUNK), :],
                dst_ref=comm_ref.at[pl.ds(src * CHUNK, CHUNK), :],
                send_sem=send_sems.at[k - 1],
                recv_sem=recv_sems.at[src],
                device_id=(src,),
                device_id_type=pl.DeviceIdType.MESH,
            )
            recv.wait_recv()
            acc = acc + comm_ref[pl.ds(src * CHUNK, CHUNK), :]
        out_ref[:, :] = acc

        for rdma in sends:
            rdma.wait_send()

    return pl.pallas_call(
        body,
        out_shape=jax.ShapeDtypeStruct((CHUNK, D_OUT), jnp.float32),
        in_specs=[
            pl.BlockSpec(memory_space=pltpu.VMEM),
            pl.BlockSpec(memory_space=pltpu.VMEM),
            pl.BlockSpec(memory_space=pltpu.VMEM),
            pl.BlockSpec(memory_space=pltpu.VMEM),
        ],
        out_specs=pl.BlockSpec(memory_space=pltpu.VMEM),
        scratch_shapes=[
            pltpu.VMEM((N_TOK, 2 * D_IN), jnp.float32),
            pltpu.VMEM((N_TOK, D_OUT), jnp.float32),
            pltpu.VMEM((N_TOK, D_OUT), jnp.float32),
            pltpu.SemaphoreType.DMA((N_DEV - 1,)),
            pltpu.SemaphoreType.DMA((N_DEV,)),
        ],
        compiler_params=pltpu.CompilerParams(collective_id=0),
    )(x, router_W, route_idx, expert_W)


# device time: 11137 ns/iter; 1.3843x vs baseline; 1.3843x over previous
import jax
import jax.numpy as jnp
from jax import lax
from jax.experimental import pallas as pl
from jax.experimental.pallas import tpu as pltpu

N_DEV = 4
N_TOK = 512
D_IN = 256
D_OUT = 512
N_EXP = 8
EXP_PER = N_EXP // N_DEV
CHUNK = N_TOK // N_DEV


def kernel(x, router_W, route_idx, expert_W):
    x = pltpu.with_memory_space_constraint(x, pltpu.MemorySpace.HBM)
    rwT = router_W.T
    expert_W = pltpu.with_memory_space_constraint(expert_W, pltpu.MemorySpace.HBM)

    def body(x_hbm, rw_ref, idx_ref, ew_hbm, out_hbm,
             x_vm, ew_vm, out_vm,
             xb_ref, wg_ref, send_buf, comm_ref,
             load_sems, store_sem, send_sems, recv_sems):
        my = lax.axis_index("i")

        ld_x = pltpu.make_async_copy(x_hbm, x_vm, load_sems.at[0])
        ld_ew = pltpu.make_async_copy(ew_hbm, ew_vm, load_sems.at[1])
        ld_x.start()
        ld_ew.start()

        barrier_sem = pltpu.get_barrier_semaphore()
        for k in range(1, N_DEV):
            pl.semaphore_signal(
                barrier_sem, inc=1,
                device_id=(jnp.mod(my + k, N_DEV),),
                device_id_type=pl.DeviceIdType.MESH,
            )

        ld_x.wait()
        xb_ref[:, :] = x_vm[:, :].astype(jnp.bfloat16)
        scores = lax.dot_general(
            xb_ref[:, :], rw_ref[:, :].astype(jnp.bfloat16),
            (((1,), (1,)), ((), ())),
            preferred_element_type=jnp.float32,
        )
        m = jnp.max(scores, axis=-1, keepdims=True)
        p = jnp.exp(scores - m)
        p = p / jnp.sum(p, axis=-1, keepdims=True)

        idx = idx_ref[:, :]
        cols = lax.broadcasted_iota(jnp.int32, (N_TOK, N_EXP), 1)
        sel0 = idx[:, 0:1] == cols
        sel1 = idx[:, 1:2] == cols
        g0 = jnp.sum(jnp.where(sel0, p, 0.0), axis=1, keepdims=True)
        g1 = jnp.sum(jnp.where(sel1, p, 0.0), axis=1, keepdims=True)
        wfull = jnp.where(sel0 | sel1, p, 0.0) / (g0 + g1)

        e_base = my * EXP_PER
        wg_ref[:, 0:1] = jnp.sum(
            jnp.where(cols == e_base, wfull, 0.0), axis=1, keepdims=True
        )
        wg_ref[:, 1:2] = jnp.sum(
            jnp.where(cols == e_base + 1, wfull, 0.0), axis=1, keepdims=True
        )

        ld_ew.wait()
        W0 = ew_vm[0].astype(jnp.bfloat16)
        W1 = ew_vm[1].astype(jnp.bfloat16)

        pl.semaphore_wait(barrier_sem, N_DEV - 1)

        def chunk_out(rows):
            y0 = jnp.dot(xb_ref[rows, :], W0, preferred_element_type=jnp.float32)
            y1 = jnp.dot(xb_ref[rows, :], W1, preferred_element_type=jnp.float32)
            return wg_ref[rows, 0:1] * y0 + wg_ref[rows, 1:2] * y1

        sends = []
        for k in range(1, N_DEV):
            dst = jnp.mod(my + k, N_DEV)
            rows = pl.ds(dst * CHUNK, CHUNK)
            send_buf[rows, :] = chunk_out(rows).astype(jnp.bfloat16)
            rdma = pltpu.make_async_remote_copy(
                src_ref=send_buf.at[rows, :],
                dst_ref=comm_ref.at[pl.ds(my * CHUNK, CHUNK), :],
                send_sem=send_sems.at[k - 1],
                recv_sem=recv_sems.at[my],
                device_id=(dst,),
                device_id_type=pl.DeviceIdType.MESH,
            )
            rdma.start()
            sends.append(rdma)

        acc = chunk_out(pl.ds(my * CHUNK, CHUNK))

        for k in range(1, N_DEV):
            src = jnp.mod(my - k, N_DEV)
            recv = pltpu.make_async_remote_copy(
                src_ref=send_buf.at[pl.ds(0, CHUNK), :],
                dst_ref=comm_ref.at[pl.ds(src * CHUNK, CHUNK), :],
                send_sem=send_sems.at[k - 1],
                recv_sem=recv_sems.at[src],
                device_id=(src,),
                device_id_type=pl.DeviceIdType.MESH,
            )
            recv.wait_recv()
            acc = acc + comm_ref[pl.ds(src * CHUNK, CHUNK), :].astype(jnp.float32)
        out_vm[:, :] = acc

        st = pltpu.make_async_copy(out_vm, out_hbm, store_sem)
        st.start()
        st.wait()

        for rdma in sends:
            rdma.wait_send()

    return pl.pallas_call(
        body,
        out_shape=jax.ShapeDtypeStruct((CHUNK, D_OUT), jnp.float32),
        in_specs=[
            pl.BlockSpec(memory_space=pltpu.MemorySpace.HBM),
            pl.BlockSpec(memory_space=pltpu.VMEM),
            pl.BlockSpec(memory_space=pltpu.VMEM),
            pl.BlockSpec(memory_space=pltpu.MemorySpace.HBM),
        ],
        out_specs=pl.BlockSpec(memory_space=pltpu.MemorySpace.HBM),
        scratch_shapes=[
            pltpu.VMEM((N_TOK, D_IN), jnp.float32),
            pltpu.VMEM((EXP_PER, D_IN, D_OUT), jnp.float32),
            pltpu.VMEM((CHUNK, D_OUT), jnp.float32),
            pltpu.VMEM((N_TOK, D_IN), jnp.bfloat16),
            pltpu.VMEM((N_TOK, 2), jnp.float32),
            pltpu.VMEM((N_TOK, D_OUT), jnp.bfloat16),
            pltpu.VMEM((N_TOK, D_OUT), jnp.bfloat16),
            pltpu.SemaphoreType.DMA((2,)),
            pltpu.SemaphoreType.DMA,
            pltpu.SemaphoreType.DMA((N_DEV - 1,)),
            pltpu.SemaphoreType.DMA((N_DEV,)),
        ],
        compiler_params=pltpu.CompilerParams(collective_id=0),
    )(x, rwT, route_idx, expert_W)
